# Initial kernel scaffold; baseline (speedup 1.0000x reference)
import jax
import jax.numpy as jnp
from jax.experimental import pallas as pl


def kernel(weight, index):
    raise NotImplementedError("write your pallas kernel here")


# speedup vs baseline: 11.1340x; 11.1340x over previous
import functools

import jax
import jax.numpy as jnp
from jax import lax
from jax.experimental import pallas as pl
from jax.experimental.pallas import tpu as pltpu
from jax.experimental.pallas import tpu_sc as plsc

_L = 17
_D = 16
_CHUNK = 2048


@functools.lru_cache(maxsize=None)
def _build_lut_gather(n_rows: int):
    info = plsc.get_sparse_core_info()
    nc, ns = info.num_cores, info.num_subcores
    nw = nc * ns
    assert n_rows % (nw * _CHUNK) == 0
    b_per_w = n_rows // nw
    n_chunks = b_per_w // _CHUNK

    mesh = plsc.VectorSubcoreMesh(core_axis_name="c", subcore_axis_name="s")

    @functools.partial(
        pl.kernel,
        mesh=mesh,
        out_type=jax.ShapeDtypeStruct((n_rows, _D), jnp.float32),
        scratch_types=[
            pltpu.VMEM((_CHUNK * 4,), jnp.int32),
            pltpu.VMEM((_CHUNK,), jnp.int32),
            pltpu.VMEM((_CHUNK, _D), jnp.float32),
            pltpu.SemaphoreType.DMA,
        ],
        compiler_params=pltpu.CompilerParams(
            needs_layout_passes=False, use_tc_tiling_on_sc=False
        ),
    )
    def lut_gather(table_hbm, idx_hbm, out_hbm, idxc, flat, rows, sem):
        wid = lax.axis_index("s") * nc + lax.axis_index("c")
        base = wid * b_per_w

        def chunk_body(t, carry):
            cbase = base + t * _CHUNK
            pltpu.sync_copy(idx_hbm.at[pl.ds(cbase * 4, _CHUNK * 4)], idxc)

            def flat_body(g, carry2):
                lanes = lax.iota(jnp.int32, 16) * 4 + g * 64
                i0 = plsc.load_gather(idxc, [lanes])
                i1 = plsc.load_gather(idxc, [lanes + 1])
                i2 = plsc.load_gather(idxc, [lanes + 2])
                i3 = plsc.load_gather(idxc, [lanes + 3])
                f = ((i0 * _L + i1) * _L + i2) * _L + i3
                flat[pl.ds(g * 16, 16)] = f
                return carry2

            lax.fori_loop(0, _CHUNK // 16, flat_body, 0)
            pltpu.async_copy(table_hbm.at[flat], rows, sem).wait()
            pltpu.sync_copy(rows, out_hbm.at[pl.ds(cbase, _CHUNK)])
            return carry

        lax.fori_loop(0, n_chunks, chunk_body, 0)

    return lut_gather


def kernel(weight, index):
    n = index.shape[0]
    up = weight.shape[-1]
    table = weight.reshape(-1, up * up)
    idx_flat = index.astype(jnp.int32).reshape(-1)
    out = _build_lut_gather(n)(table, idx_flat)
    return out.reshape(n, up, up)


# speedup vs baseline: 54.1788x; 4.8661x over previous
import functools

import jax
import jax.numpy as jnp
from jax import lax
from jax.experimental import pallas as pl
from jax.experimental.pallas import tpu as pltpu
from jax.experimental.pallas import tpu_sc as plsc

_L = 17
_JL = 8


@functools.lru_cache(maxsize=None)
def _build_lut_gather(n_rows: int):
    info = plsc.get_sparse_core_info()
    nc, ns = info.num_cores, info.num_subcores
    nw = nc * ns
    chunk = _JL * 128
    assert n_rows % (nw * chunk) == 0
    jb = n_rows // 128
    jw = jb // nw
    n_chunks = jw // _JL
    out_plane = jb * 512

    mesh = plsc.VectorSubcoreMesh(core_axis_name="c", subcore_axis_name="s")

    @functools.partial(
        pl.kernel,
        mesh=mesh,
        out_type=jax.ShapeDtypeStruct((4 * out_plane,), jnp.float32),
        scratch_types=[
            pltpu.VMEM((chunk * 4,), jnp.int32),
            pltpu.VMEM((chunk,), jnp.int32),
            pltpu.VMEM((chunk, 16), jnp.float32),
            pltpu.VMEM((chunk * 4,), jnp.float32),
            pltpu.VMEM((chunk * 4,), jnp.float32),
            pltpu.VMEM((chunk * 4,), jnp.float32),
            pltpu.VMEM((chunk * 4,), jnp.float32),
            pltpu.SemaphoreType.DMA,
        ],
        compiler_params=pltpu.CompilerParams(
            needs_layout_passes=False, use_tc_tiling_on_sc=False
        ),
    )
    def lut_gather(table_hbm, idx_hbm, out_hbm, idxc, flat, rows,
                   t0, t1, t2, t3, sem):
        wid = lax.axis_index("s") * nc + lax.axis_index("c")
        base_j = wid * jw
        trsp = (t0, t1, t2, t3)

        def chunk_body(t, carry):
            j0 = base_j + t * _JL
            pltpu.sync_copy(idx_hbm.at[pl.ds(j0 * 512, chunk * 4)], idxc)

            def flat_body(g, carry2):
                jl = g // 8
                k = g % 8
                s = jl * 512 + k * 16
                i0 = idxc[pl.ds(s, 16)]
                i1 = idxc[pl.ds(s + 128, 16)]
                i2 = idxc[pl.ds(s + 256, 16)]
                i3 = idxc[pl.ds(s + 384, 16)]
                f = ((i0 * _L + i1) * _L + i2) * _L + i3
                flat[pl.ds(jl * 128 + k * 16, 16)] = f
                return carry2

            lax.fori_loop(0, _JL * 8, flat_body, 0)
            pltpu.async_copy(table_hbm.at[flat], rows, sem).wait()

            for u in range(4):
                for v in range(4):
                    col = jnp.full((16,), 4 * u + v, jnp.int32)

                    def tr_body(g, carry2, col=col, dst=trsp[u], v=v):
                        jl = g // 8
                        k = g % 8
                        r = jl * 128 + k * 16 + lax.iota(jnp.int32, 16)
                        val = plsc.load_gather(rows, [r, col])
                        dst[pl.ds((jl * 4 + v) * 128 + k * 16, 16)] = val
                        return carry2

                    lax.fori_loop(0, _JL * 8, tr_body, 0)

            for u in range(4):
                pltpu.sync_copy(
                    trsp[u],
                    out_hbm.at[pl.ds(u * out_plane + j0 * 512, chunk * 4)],
                )
            return carry

        lax.fori_loop(0, n_chunks, chunk_body, 0)

    return lut_gather


def kernel(weight, index):
    n = index.shape[0]
    up = weight.shape[-1]
    table = weight.reshape(-1, up * up)
    idxv = (
        index.astype(jnp.int32)
        .reshape(n // 128, 128, 4)
        .transpose(0, 2, 1)
        .reshape(n * 4)
    )
    o = _build_lut_gather(n)(table, idxv)
    return (
        o.reshape(4, n // 128, 4, 128)
        .transpose(1, 3, 0, 2)
        .reshape(n, up, up)
    )
